# baseline (speedup 1.0000x reference)
import functools

import jax
import jax.numpy as jnp
from jax import lax
from jax.experimental import pallas as pl
from jax.experimental.pallas import tpu as pltpu
from jax.experimental.pallas import tpu_sc as plsc

NC, NS = 2, 16
NW = NC * NS
BATCH, SEQ = 4, 4096
N = BATCH * SEQ
D = 1024
PER_W = N // NW
CHUNK = 32
NCHUNK = PER_W // CHUNK

_mesh = plsc.VectorSubcoreMesh(
    core_axis_name="c", subcore_axis_name="s", num_cores=NC, num_subcores=NS
)


@functools.partial(
    pl.kernel,
    out_type=jax.ShapeDtypeStruct((N, D), jnp.float32),
    mesh=_mesh,
    scratch_types=[
        pltpu.VMEM((NCHUNK, CHUNK), jnp.int32),
        pltpu.VMEM((CHUNK, D), jnp.float32),
        pltpu.VMEM((CHUNK, D), jnp.float32),
        pltpu.VMEM((CHUNK, D), jnp.float32),
        pltpu.SemaphoreType.DMA,
        pltpu.SemaphoreType.DMA,
    ],
)
def _emb_lookup(idx_hbm, table_hbm, out_hbm, idx_v, rows0, rows1, rows2, gsem, ssem):
    wid = lax.axis_index("s") * NC + lax.axis_index("c")
    base = wid * PER_W
    pltpu.sync_copy(idx_hbm.at[wid], idx_v)

    bufs = (rows0, rows1, rows2)
    nbuf = len(bufs)
    gathers = [None] * NCHUNK
    scatters = [None] * NCHUNK
    for j in range(nbuf - 1):
        gathers[j] = pltpu.async_copy(table_hbm.at[idx_v.at[j]], bufs[j], gsem)
    for j in range(NCHUNK):
        if j + nbuf - 1 < NCHUNK:
            if j >= 1:
                scatters[j - 1].wait()
            jn = j + nbuf - 1
            gathers[jn] = pltpu.async_copy(
                table_hbm.at[idx_v.at[jn]], bufs[jn % nbuf], gsem
            )
        gathers[j].wait()
        scatters[j] = pltpu.async_copy(
            bufs[j % nbuf], out_hbm.at[pl.ds(base + j * CHUNK, CHUNK)], ssem
        )
    for j in range(NCHUNK - nbuf, NCHUNK):
        scatters[j].wait()


def kernel(position_ids, pos_emb_weight):
    ids = position_ids.astype(jnp.int32).reshape(NW, NCHUNK, CHUNK)
    out = _emb_lookup(ids, pos_emb_weight)
    return out.reshape(BATCH, SEQ, D)
